# baseline (speedup 1.0000x reference)
import jax
import jax.numpy as jnp
from jax.experimental import pallas as pl
from jax.experimental.pallas import tpu as pltpu

D_MODEL = 4096
NUM_EXPERTS = 64
BLOCK_M = 512


def _router_kernel(idx_ref, x_ref, w_ref, o_ref):
    del idx_ref
    x = x_ref[...]
    w = w_ref[0]
    o_ref[...] = jax.lax.dot_general(
        x, w, (((1,), (1,)), ((), ())), preferred_element_type=jnp.float32
    )


def kernel(index, hidden_states, W):
    m = hidden_states.shape[0]
    idx = jnp.asarray(index, dtype=jnp.int32).reshape((1,))
    grid_spec = pltpu.PrefetchScalarGridSpec(
        num_scalar_prefetch=1,
        grid=(m // BLOCK_M,),
        in_specs=[
            pl.BlockSpec((BLOCK_M, D_MODEL), lambda i, idx_ref: (i, 0)),
            pl.BlockSpec(
                (1, NUM_EXPERTS, D_MODEL), lambda i, idx_ref: (idx_ref[0], 0, 0)
            ),
        ],
        out_specs=pl.BlockSpec((BLOCK_M, NUM_EXPERTS), lambda i, idx_ref: (i, 0)),
    )
    return pl.pallas_call(
        _router_kernel,
        grid_spec=grid_spec,
        out_shape=jax.ShapeDtypeStruct((m, NUM_EXPERTS), jnp.float32),
    )(idx, hidden_states, W)


# speedup vs baseline: 1.0015x; 1.0015x over previous
import jax
import jax.numpy as jnp
from jax.experimental import pallas as pl
from jax.experimental.pallas import tpu as pltpu

D_MODEL = 4096
NUM_EXPERTS = 64
BLOCK_M = 1024


def _router_kernel(idx_ref, x_ref, w_ref, o_ref):
    del idx_ref
    x = x_ref[...]
    w = w_ref[0]
    o_ref[...] = jax.lax.dot_general(
        x, w, (((1,), (1,)), ((), ())), preferred_element_type=jnp.float32
    )


def kernel(index, hidden_states, W):
    m = hidden_states.shape[0]
    idx = jnp.asarray(index, dtype=jnp.int32).reshape((1,))
    grid_spec = pltpu.PrefetchScalarGridSpec(
        num_scalar_prefetch=1,
        grid=(m // BLOCK_M,),
        in_specs=[
            pl.BlockSpec((BLOCK_M, D_MODEL), lambda i, idx_ref: (i, 0)),
            pl.BlockSpec(
                (1, NUM_EXPERTS, D_MODEL), lambda i, idx_ref: (idx_ref[0], 0, 0)
            ),
        ],
        out_specs=pl.BlockSpec((BLOCK_M, NUM_EXPERTS), lambda i, idx_ref: (i, 0)),
    )
    return pl.pallas_call(
        _router_kernel,
        grid_spec=grid_spec,
        out_shape=jax.ShapeDtypeStruct((m, NUM_EXPERTS), jnp.float32),
        compiler_params=pltpu.CompilerParams(
            dimension_semantics=("parallel",),
        ),
    )(idx, hidden_states, W)


# speedup vs baseline: 1.0015x; 1.0000x over previous
import jax
import jax.numpy as jnp
from jax.experimental import pallas as pl
from jax.experimental.pallas import tpu as pltpu

D_MODEL = 4096
NUM_EXPERTS = 64
BLOCK_M = 512
X_BUFFERS = 4


def _router_kernel(idx_ref, x_hbm, w_ref, o_hbm):
    del idx_ref

    def body(x_ref, o_ref):
        o_ref[...] = jax.lax.dot_general(
            x_ref[...],
            w_ref[0],
            (((1,), (1,)), ((), ())),
            preferred_element_type=jnp.float32,
        )

    m = x_hbm.shape[0]
    pipeline = pltpu.emit_pipeline(
        body,
        grid=(m // BLOCK_M,),
        in_specs=[
            pl.BlockSpec(
                (BLOCK_M, D_MODEL),
                lambda i: (i, 0),
                pipeline_mode=pl.Buffered(buffer_count=X_BUFFERS),
            )
        ],
        out_specs=[pl.BlockSpec((BLOCK_M, NUM_EXPERTS), lambda i: (i, 0))],
    )
    pipeline(x_hbm, o_hbm)


def kernel(index, hidden_states, W):
    m = hidden_states.shape[0]
    idx = jnp.asarray(index, dtype=jnp.int32).reshape((1,))
    grid_spec = pltpu.PrefetchScalarGridSpec(
        num_scalar_prefetch=1,
        grid=(1,),
        in_specs=[
            pl.BlockSpec(memory_space=pl.ANY),
            pl.BlockSpec(
                (1, NUM_EXPERTS, D_MODEL), lambda i, idx_ref: (idx_ref[0], 0, 0)
            ),
        ],
        out_specs=pl.BlockSpec(memory_space=pl.ANY),
    )
    return pl.pallas_call(
        _router_kernel,
        grid_spec=grid_spec,
        out_shape=jax.ShapeDtypeStruct((m, NUM_EXPERTS), jnp.float32),
    )(idx, hidden_states, W)
